# baseline (speedup 1.0000x reference)
import functools

import jax
import jax.numpy as jnp
from jax import lax
from jax.experimental import pallas as pl
from jax.experimental.pallas import tpu as pltpu
from jax.experimental.pallas import tpu_sc as plsc

VOCAB = 1000000
D = 64
B = 4096
S = 200

_INFO = plsc.get_sparse_core_info()
_NC, _NS = _INFO.num_cores, _INFO.num_subcores
_NW = _NC * _NS

_IW = 128
_NROWS = (B * S) // _IW
_RPW = _NROWS // _NW
_KCH = 8
_NGRP = _RPW // _KCH


def _make_gather():
    mesh = plsc.VectorSubcoreMesh(core_axis_name="c", subcore_axis_name="s")

    @functools.partial(
        pl.kernel,
        mesh=mesh,
        out_type=jax.ShapeDtypeStruct((_NROWS, _IW, D), jnp.float32),
        scratch_types=[
            pltpu.VMEM((_RPW, _IW), jnp.int32),
            pltpu.VMEM((_KCH, _IW, D), jnp.float32),
            pltpu.SemaphoreType.DMA,
        ],
        compiler_params=pltpu.CompilerParams(use_tc_tiling_on_sc=False),
    )
    def gather_kernel(table_hbm, idx_hbm, out_hbm, idx_v, rows_v, sem):
        wid = lax.axis_index("s") * _NC + lax.axis_index("c")
        rbase = wid * _RPW
        pltpu.sync_copy(idx_hbm.at[pl.ds(rbase, _RPW)], idx_v)

        def body(g, carry):
            handles = []
            for j in range(_KCH):
                handles.append(
                    pltpu.async_copy(
                        table_hbm.at[idx_v.at[g * _KCH + j]],
                        rows_v.at[j],
                        sem,
                    )
                )
            for h in handles:
                h.wait()
            pltpu.sync_copy(rows_v, out_hbm.at[pl.ds(rbase + g * _KCH, _KCH)])
            return carry

        lax.fori_loop(0, _NGRP, body, 0)

    return gather_kernel


_gather = _make_gather()


def kernel(tokens, embedding):
    idx = tokens.astype(jnp.int32).reshape(_NROWS, _IW)
    out = _gather(embedding, idx)
    return out.reshape(B, S, D)


# speedup vs baseline: 1.0085x; 1.0085x over previous
import functools

import jax
import jax.numpy as jnp
from jax import lax
from jax.experimental import pallas as pl
from jax.experimental.pallas import tpu as pltpu
from jax.experimental.pallas import tpu_sc as plsc

VOCAB = 1000000
D = 64
B = 4096
S = 200

_INFO = plsc.get_sparse_core_info()
_NC, _NS = _INFO.num_cores, _INFO.num_subcores
_NW = _NC * _NS

_IW = 128
_NROWS = (B * S) // _IW
_RPW = _NROWS // _NW
_KCH = 4
_NGRP = _RPW // _KCH
_HALF = _NGRP // 2


def _make_gather():
    mesh = plsc.VectorSubcoreMesh(core_axis_name="c", subcore_axis_name="s")

    @functools.partial(
        pl.kernel,
        mesh=mesh,
        out_type=jax.ShapeDtypeStruct((_NROWS, _IW, D), jnp.float32),
        scratch_types=[
            pltpu.VMEM((_RPW, _IW), jnp.int32),
            pltpu.VMEM((_KCH, _IW, D), jnp.float32),
            pltpu.VMEM((_KCH, _IW, D), jnp.float32),
            pltpu.SemaphoreType.DMA,
            pltpu.SemaphoreType.DMA,
            pltpu.SemaphoreType.DMA,
            pltpu.SemaphoreType.DMA,
        ],
        compiler_params=pltpu.CompilerParams(use_tc_tiling_on_sc=False),
    )
    def gather_kernel(table_hbm, idx_hbm, out_hbm, idx_v, rows0, rows1,
                      gs0, gs1, ws0, ws1):
        wid = lax.axis_index("s") * _NC + lax.axis_index("c")
        rbase = wid * _RPW
        pltpu.sync_copy(idx_hbm.at[pl.ds(rbase, _RPW)], idx_v)

        def fire_gathers(buf, sem, g):
            for j in range(_KCH):
                pltpu.async_copy(table_hbm.at[idx_v.at[g * _KCH + j]],
                                 buf.at[j], sem)

        def wait_gathers(buf, sem):
            pltpu.make_async_copy(out_hbm.at[pl.ds(0, _KCH)], buf, sem).wait()

        def fire_write(buf, sem, g):
            pltpu.async_copy(buf, out_hbm.at[pl.ds(rbase + g * _KCH, _KCH)],
                             sem)

        def wait_write(buf, sem):
            pltpu.make_async_copy(buf, out_hbm.at[pl.ds(rbase, _KCH)],
                                  sem).wait()

        fire_gathers(rows0, gs0, 0)

        def body(t, carry):
            g0 = 2 * t
            wait_gathers(rows0, gs0)

            @pl.when(t >= 1)
            def _():
                wait_write(rows1, ws1)

            fire_write(rows0, ws0, g0)
            fire_gathers(rows1, gs1, g0 + 1)
            wait_gathers(rows1, gs1)
            wait_write(rows0, ws0)
            fire_write(rows1, ws1, g0 + 1)

            @pl.when(t <= _HALF - 2)
            def _():
                fire_gathers(rows0, gs0, g0 + 2)

            return carry

        lax.fori_loop(0, _HALF, body, 0)
        wait_write(rows1, ws1)

    return gather_kernel


_gather = _make_gather()


def kernel(tokens, embedding):
    idx = tokens.astype(jnp.int32).reshape(_NROWS, _IW)
    out = _gather(embedding, idx)
    return out.reshape(B, S, D)
